# baseline (speedup 1.0000x reference)
import functools

import jax
import jax.numpy as jnp
from jax import lax
from jax.experimental import pallas as pl
from jax.experimental.pallas import tpu as pltpu
from jax.experimental.pallas import tpu_sc as plsc

N = 5000
D = 256
NP = 5120
B = 512
NB = NP // B
THRESH = 0.5
MROWS = 1000


def _nms_body(cols_ref, rows_ref, sr_ref, sc_ref, keep_ref,
              cC_ref, aC_ref, cR_ref, aR_ref):
    cC_ref[...] = cols_ref[...] * sr_ref[...]
    cR_ref[...] = rows_ref[...] * sc_ref[...]
    aC_ref[...] = ((cC_ref[:, 2:3] - cC_ref[:, 0:1]) *
                   (cC_ref[:, 3:4] - cC_ref[:, 1:2]))
    aR_ref[...] = ((cR_ref[2:3, :] - cR_ref[0:1, :]) *
                   (cR_ref[3:4, :] - cR_ref[1:2, :]))
    keep_ref[...] = jnp.ones_like(keep_ref)
    ri = jax.lax.broadcasted_iota(jnp.int32, (B, B), 0)
    ci = jax.lax.broadcasted_iota(jnp.int32, (B, B), 1)
    tri = ri < ci

    def tile(bx1, by1, bx2, by2, barea, j0, diag):
        x1b = cR_ref[0:1, pl.ds(j0, B)]
        y1b = cR_ref[1:2, pl.ds(j0, B)]
        x2b = cR_ref[2:3, pl.ds(j0, B)]
        y2b = cR_ref[3:4, pl.ds(j0, B)]
        ab = aR_ref[0:1, pl.ds(j0, B)]
        wx = jnp.maximum(
            jnp.minimum(bx2, x2b) - jnp.maximum(bx1, x1b), 0.0)
        wy = jnp.maximum(
            jnp.minimum(by2, y2b) - jnp.maximum(by1, y1b), 0.0)
        inter = wx * wy
        union = (barea + ab) - inter
        iou = inter / jnp.maximum(union, 1e-9)
        ov = iou > THRESH
        if diag:
            ov = ov & tri
        return ov.astype(jnp.float32)

    def iblock(i, carry):
        i0 = i * B
        zz = jnp.zeros((B, B), jnp.float32)
        bx1 = cC_ref[pl.ds(i0, B), 0:1] + zz
        by1 = cC_ref[pl.ds(i0, B), 1:2] + zz
        bx2 = cC_ref[pl.ds(i0, B), 2:3] + zz
        by2 = cC_ref[pl.ds(i0, B), 3:4] + zz
        barea = aC_ref[pl.ds(i0, B), 0:1] + zz

        a_ii = tile(bx1, by1, bx2, by2, barea, i0, True)
        inc = keep_ref[0:1, pl.ds(i0, B)]

        def cond(c):
            return c[1]

        def body(c):
            v = c[0]
            sup = jax.lax.dot_general(
                v, a_ii, (((1,), (0,)), ((), ())),
                preferred_element_type=jnp.float32)
            vn = inc * (1.0 - (sup > 0.0).astype(jnp.float32))
            return vn, jnp.sum(jnp.abs(vn - v)) > 0.0

        vfin, _ = jax.lax.while_loop(cond, body, (inc, jnp.bool_(True)))
        keep_ref[0:1, pl.ds(i0, B)] = vfin

        def jblock(j, c2):
            j0 = j * B
            a_ij = tile(bx1, by1, bx2, by2, barea, j0, False)
            sup = jax.lax.dot_general(
                vfin, a_ij, (((1,), (0,)), ((), ())),
                preferred_element_type=jnp.float32)
            cur = keep_ref[0:1, pl.ds(j0, B)]
            keep_ref[0:1, pl.ds(j0, B)] = cur * (
                1.0 - (sup > 0.0).astype(jnp.float32))
            return c2

        jax.lax.fori_loop(i + 1, NB, jblock, 0)
        return carry

    jax.lax.fori_loop(0, NB, iblock, 0)


_SC_MESH = plsc.VectorSubcoreMesh(core_axis_name="c", subcore_axis_name="s")


_NWORK = 32
_CH = NP // _NWORK
_SUB = _CH // 2


@functools.partial(
    pl.kernel,
    out_type=jax.ShapeDtypeStruct((NP,), jnp.float32),
    mesh=_SC_MESH,
    scratch_types=[
        pltpu.VMEM((_SUB,), jnp.float32),
        pltpu.VMEM((_SUB,), jnp.int32),
        pltpu.VMEM((_SUB,), jnp.float32),
        pltpu.VMEM((_SUB,), jnp.int32),
        pltpu.SemaphoreType.DMA,
    ],
)
def _sc_unpermute(keep_hbm, order_hbm, out_hbm, v0, ix0, v1, ix1, sem):
    wid = lax.axis_index("s") * 2 + lax.axis_index("c")
    base = wid * _CH
    pltpu.sync_copy(keep_hbm.at[pl.ds(base, _SUB)], v0)
    pltpu.sync_copy(order_hbm.at[pl.ds(base, _SUB)], ix0)
    pltpu.sync_copy(keep_hbm.at[pl.ds(base + _SUB, _SUB)], v1)
    pltpu.sync_copy(order_hbm.at[pl.ds(base + _SUB, _SUB)], ix1)
    pltpu.async_copy(v0, out_hbm.at[ix0], sem).wait()
    pltpu.async_copy(v1, out_hbm.at[ix1], sem).wait()


def _mask_body(feat_ref, rs_ref, keep_ref, scale_ref, fo_ref, ro_ref):
    k = keep_ref[...]
    fo_ref[...] = feat_ref[...] * k
    ro_ref[...] = (rs_ref[...] * scale_ref[...]) * k


def kernel(features, rois, scores, scale_fct):
    order = jnp.argsort(-scores)
    rois8 = jnp.pad(rois, ((0, NP - N), (0, 4)))
    order_pad = jnp.concatenate(
        [order, jnp.arange(N, NP, dtype=order.dtype)])
    cols = rois8[order_pad]
    rows = cols.T
    scale8 = jnp.concatenate(
        [scale_fct[0], jnp.zeros((4,), jnp.float32)])[None]
    scale8c = scale8.T

    keep8 = pl.pallas_call(
        _nms_body,
        out_shape=jax.ShapeDtypeStruct((8, NP), jnp.float32),
        scratch_shapes=[
            pltpu.VMEM((NP, 8), jnp.float32),
            pltpu.VMEM((NP, 1), jnp.float32),
            pltpu.VMEM((8, NP), jnp.float32),
            pltpu.VMEM((1, NP), jnp.float32),
        ],
    )(cols, rows, scale8, scale8c)

    keep_orig = _sc_unpermute(keep8[0], order_pad)
    keep_f = keep_orig[:N, None]
    keep = keep_f[:, 0] > 0.5

    rs = jnp.concatenate(
        [rois, scores[:, None], jnp.zeros((N, 3), jnp.float32)], axis=1)
    scale5 = jnp.concatenate(
        [scale_fct[0], jnp.ones((1,), jnp.float32),
         jnp.zeros((3,), jnp.float32)])[None]

    feats_out, rs_out = pl.pallas_call(
        _mask_body,
        grid=(N // MROWS,),
        in_specs=[
            pl.BlockSpec((MROWS, D), lambda i: (i, 0)),
            pl.BlockSpec((MROWS, 8), lambda i: (i, 0)),
            pl.BlockSpec((MROWS, 1), lambda i: (i, 0)),
            pl.BlockSpec((1, 8), lambda i: (0, 0)),
        ],
        out_specs=(
            pl.BlockSpec((MROWS, D), lambda i: (i, 0)),
            pl.BlockSpec((MROWS, 8), lambda i: (i, 0)),
        ),
        out_shape=(
            jax.ShapeDtypeStruct((N, D), jnp.float32),
            jax.ShapeDtypeStruct((N, 8), jnp.float32),
        ),
    )(features, rs, keep_f, scale5)

    rois_out = rs_out[:, :5]
    return feats_out, rois_out, keep
